# baseline (speedup 1.0000x reference)
import functools

import numpy as np
import jax
import jax.numpy as jnp
from jax.experimental import pallas as pl
from jax.experimental.pallas import tpu as pltpu


def _build_tap_mask(H2, W2):
    m = np.zeros((9, H2 * W2), np.float32)
    for dh in range(3):
        for dw in range(3):
            t = dh * 3 + dw
            ii = np.arange(H2)[:, None] + dh - 1
            jj = np.arange(W2)[None, :] + dw - 1
            valid = (ii >= 0) & (ii < H2) & (jj >= 0) & (jj < W2)
            m[t] = valid.astype(np.float32).reshape(-1)
    return jnp.asarray(m)


def _build_scatter(H, W):
    H2, W2 = 2 * H, 2 * W
    E = np.zeros((4, H * W, H2 * W2), np.float32)
    ii = np.arange(H2)
    jj = np.arange(W2)
    for i in range(H2):
        for j in range(W2):
            p = (i % 2) * 2 + (j % 2)
            E[p, (i // 2) * W + (j // 2), i * W2 + j] = 1.0
    return jnp.asarray(E, jnp.bfloat16)


def _unet_fused_kernel(u_ref, x_ref, wstack_ref, bup_ref, e_ref,
                       w1_ref, b1_ref, w2_ref, b2_ref, tmask_ref,
                       out_ref, sum_ref, ssq_ref, *, W2, Co, Cout, NB):
    HW2 = x_ref.shape[-1]
    masks = tmask_ref[...]

    def conv3x3_relu(xb, w_all, b_col, C):
        y = jnp.dot(w_all, xb,
                    preferred_element_type=jnp.float32)
        acc = y[4 * C:5 * C] + b_col
        for t in range(9):
            if t == 4:
                continue
            dh, dw = t // 3, t % 3
            off = (dh - 1) * W2 + (dw - 1)
            shifted = pltpu.roll(y[t * C:(t + 1) * C],
                                 shift=(-off) % HW2, axis=1)
            acc = acc + shifted * masks[t:t + 1, :]
        return jnp.maximum(acc, 0.0)

    for i in range(NB):
        ub = u_ref[i].astype(jnp.bfloat16)
        z = jnp.dot(wstack_ref[...], ub,
                    preferred_element_type=jnp.float32)
        zb = z.astype(jnp.bfloat16)
        up = jnp.dot(zb[0 * Co:1 * Co], e_ref[0],
                     preferred_element_type=jnp.float32)
        for p in range(1, 4):
            up = up + jnp.dot(zb[p * Co:(p + 1) * Co], e_ref[p],
                              preferred_element_type=jnp.float32)
        up = up + bup_ref[...]

        cat_b = jnp.concatenate([up.astype(jnp.bfloat16),
                                 x_ref[i].astype(jnp.bfloat16)], axis=0)

        h1 = conv3x3_relu(cat_b, w1_ref[...], b1_ref[...], Cout)
        h2 = conv3x3_relu(h1.astype(jnp.bfloat16), w2_ref[...], b2_ref[...],
                          Cout)

        out_ref[i] = h2.astype(jnp.bfloat16)
        sum_ref[i] = jnp.sum(h2, axis=1, keepdims=True)
        ssq_ref[i] = jnp.sum(h2 * h2, axis=1, keepdims=True)


def _bn_apply_kernel(h_ref, scale_ref, shift_ref, out_ref):
    o = h_ref[...].astype(jnp.float32) * scale_ref[...] + shift_ref[...]
    out_ref[...] = o.reshape(out_ref.shape)


@jax.jit
def _unet_block_forward(up_in, x_in, params):
    B, Cup, H, W = up_in.shape
    Co = Cup // 2
    _, Cx, H2, W2 = x_in.shape
    assert H2 == 2 * H and W2 == 2 * W
    Cmid = Cx + Co
    Cout = Cmid // 2
    HW = H * W
    HW2 = H2 * W2
    eps = 1e-5
    assert Cup % 8 == 0 and Co % 8 == 0 and Cx % 8 == 0 and Cout % 8 == 0

    u_flat = up_in.reshape(B, Cup, HW)
    x_flat = x_in.reshape(B, Cx, HW2)

    wstack = jnp.transpose(params["w_up"], (2, 3, 1, 0)) \
        .reshape(4 * Co, Cup).astype(jnp.bfloat16)
    bup = params["b_up"].reshape(Co, 1)

    w1_all = jnp.transpose(params["w1"], (2, 3, 0, 1)) \
        .reshape(9 * Cout, Cmid).astype(jnp.bfloat16)
    b1 = params["b1"].reshape(Cout, 1)
    w2_all = jnp.transpose(params["w2"], (2, 3, 0, 1)) \
        .reshape(9 * Cout, Cout).astype(jnp.bfloat16)
    b2 = params["b2"].reshape(Cout, 1)
    gamma = params["gamma"].reshape(Cout, 1)
    beta = params["beta"].reshape(Cout, 1)

    emat = _build_scatter(H, W)
    tmask = _build_tap_mask(H2, W2)

    cparams = pltpu.CompilerParams(
        dimension_semantics=("parallel",),
        vmem_limit_bytes=48 * 1024 * 1024,
    )

    NB1 = 4 if B % 8 == 0 else 1
    kernel1 = functools.partial(_unet_fused_kernel, W2=W2, Co=Co, Cout=Cout,
                                NB=NB1)
    h2, psum, psq = pl.pallas_call(
        kernel1,
        out_shape=(jax.ShapeDtypeStruct((B, Cout, HW2), jnp.bfloat16),
                   jax.ShapeDtypeStruct((B, Cout, 1), jnp.float32),
                   jax.ShapeDtypeStruct((B, Cout, 1), jnp.float32)),
        grid=(B // NB1,),
        in_specs=[
            pl.BlockSpec((NB1, Cup, HW), lambda b: (b, 0, 0)),
            pl.BlockSpec((NB1, Cx, HW2), lambda b: (b, 0, 0)),
            pl.BlockSpec((4 * Co, Cup), lambda b: (0, 0)),
            pl.BlockSpec((Co, 1), lambda b: (0, 0)),
            pl.BlockSpec((4, HW, HW2), lambda b: (0, 0, 0)),
            pl.BlockSpec((9 * Cout, Cmid), lambda b: (0, 0)),
            pl.BlockSpec((Cout, 1), lambda b: (0, 0)),
            pl.BlockSpec((9 * Cout, Cout), lambda b: (0, 0)),
            pl.BlockSpec((Cout, 1), lambda b: (0, 0)),
            pl.BlockSpec((9, HW2), lambda b: (0, 0)),
        ],
        out_specs=(pl.BlockSpec((NB1, Cout, HW2), lambda b: (b, 0, 0)),
                   pl.BlockSpec((NB1, Cout, 1), lambda b: (b, 0, 0)),
                   pl.BlockSpec((NB1, Cout, 1), lambda b: (b, 0, 0))),
        compiler_params=cparams,
    )(u_flat, x_flat, wstack, bup, emat, w1_all, b1, w2_all, b2, tmask)

    count = B * HW2
    mean = jnp.sum(psum, axis=0) / count
    var = jnp.sum(psq, axis=0) / count - mean * mean
    scale = gamma * jax.lax.rsqrt(var + eps)
    shift = beta - mean * scale

    NB = 4 if B % 4 == 0 else 1
    out = pl.pallas_call(
        _bn_apply_kernel,
        out_shape=jax.ShapeDtypeStruct((B, Cout, H2, W2), jnp.float32),
        grid=(B // NB,),
        in_specs=[pl.BlockSpec((NB, Cout, HW2), lambda b: (b, 0, 0)),
                  pl.BlockSpec((Cout, 1), lambda b: (0, 0)),
                  pl.BlockSpec((Cout, 1), lambda b: (0, 0))],
        out_specs=pl.BlockSpec((NB, Cout, H2, W2), lambda b: (b, 0, 0, 0)),
        compiler_params=cparams,
    )(h2, scale, shift)

    return out


def kernel(up_in, x_in, w_up, b_up, w1, b1, w2, b2, gamma, beta):
    params = {"w_up": w_up, "b_up": b_up, "w1": w1, "b1": b1,
              "w2": w2, "b2": b2, "gamma": gamma, "beta": beta}
    return _unet_block_forward(up_in, x_in, params)


# speedup vs baseline: 1.2451x; 1.2451x over previous
import functools

import numpy as np
import jax
import jax.numpy as jnp
from jax.experimental import pallas as pl
from jax.experimental.pallas import tpu as pltpu


def _build_tap_mask(H2, W2):
    m = np.zeros((9, H2 * W2), np.float32)
    for dh in range(3):
        for dw in range(3):
            t = dh * 3 + dw
            ii = np.arange(H2)[:, None] + dh - 1
            jj = np.arange(W2)[None, :] + dw - 1
            valid = (ii >= 0) & (ii < H2) & (jj >= 0) & (jj < W2)
            m[t] = valid.astype(np.float32).reshape(-1)
    return jnp.asarray(m)


def _build_scatter(H, W):
    H2, W2 = 2 * H, 2 * W
    E = np.zeros((4, H * W, H2 * W2), np.float32)
    ii = np.arange(H2)
    jj = np.arange(W2)
    for i in range(H2):
        for j in range(W2):
            p = (i % 2) * 2 + (j % 2)
            E[p, (i // 2) * W + (j // 2), i * W2 + j] = 1.0
    return jnp.asarray(E, jnp.bfloat16)


def _unet_fused_kernel(u_ref, x_ref, wstack_ref, bup_ref, e_ref,
                       w1_ref, b1_ref, w2_ref, b2_ref, tmask_ref,
                       out_ref, sum_ref, ssq_ref, *, W2, Co, Cout, NB):
    HW2 = x_ref.shape[-1]
    masks = tmask_ref[...]

    def conv3x3_relu(xb, w_all, b_col, C):
        y = jnp.dot(w_all, xb,
                    preferred_element_type=jnp.float32)
        acc = y[4 * C:5 * C] + b_col
        for t in range(9):
            if t == 4:
                continue
            dh, dw = t // 3, t % 3
            off = (dh - 1) * W2 + (dw - 1)
            shifted = pltpu.roll(y[t * C:(t + 1) * C],
                                 shift=(-off) % HW2, axis=1)
            acc = acc + shifted * masks[t:t + 1, :]
        return jnp.maximum(acc, 0.0)

    for i in range(NB):
        ub = u_ref[i].astype(jnp.bfloat16)
        z = jnp.dot(wstack_ref[...], ub,
                    preferred_element_type=jnp.float32)
        zb = z.astype(jnp.bfloat16)
        up = jnp.dot(zb[0 * Co:1 * Co], e_ref[0],
                     preferred_element_type=jnp.float32)
        for p in range(1, 4):
            up = up + jnp.dot(zb[p * Co:(p + 1) * Co], e_ref[p],
                              preferred_element_type=jnp.float32)
        up = up + bup_ref[...]

        cat_b = jnp.concatenate([up.astype(jnp.bfloat16),
                                 x_ref[i].astype(jnp.bfloat16)], axis=0)

        h1 = conv3x3_relu(cat_b, w1_ref[...], b1_ref[...], Cout)
        h2 = conv3x3_relu(h1.astype(jnp.bfloat16), w2_ref[...], b2_ref[...],
                          Cout)

        out_ref[i] = h2.astype(jnp.bfloat16)
        sum_ref[i] = jnp.sum(h2, axis=1, keepdims=True)
        ssq_ref[i] = jnp.sum(h2 * h2, axis=1, keepdims=True)


def _bn_apply_kernel(h_ref, scale_ref, shift_ref, out_ref):
    out_ref[...] = (h_ref[...].astype(jnp.float32) * scale_ref[...]
                    + shift_ref[...])


@jax.jit
def _unet_block_forward(up_in, x_in, params):
    B, Cup, H, W = up_in.shape
    Co = Cup // 2
    _, Cx, H2, W2 = x_in.shape
    assert H2 == 2 * H and W2 == 2 * W
    Cmid = Cx + Co
    Cout = Cmid // 2
    HW = H * W
    HW2 = H2 * W2
    eps = 1e-5
    assert Cup % 8 == 0 and Co % 8 == 0 and Cx % 8 == 0 and Cout % 8 == 0

    u_flat = up_in.reshape(B, Cup, HW)
    x_flat = x_in.reshape(B, Cx, HW2)

    wstack = jnp.transpose(params["w_up"], (2, 3, 1, 0)) \
        .reshape(4 * Co, Cup).astype(jnp.bfloat16)
    bup = params["b_up"].reshape(Co, 1)

    w1_all = jnp.transpose(params["w1"], (2, 3, 0, 1)) \
        .reshape(9 * Cout, Cmid).astype(jnp.bfloat16)
    b1 = params["b1"].reshape(Cout, 1)
    w2_all = jnp.transpose(params["w2"], (2, 3, 0, 1)) \
        .reshape(9 * Cout, Cout).astype(jnp.bfloat16)
    b2 = params["b2"].reshape(Cout, 1)
    gamma = params["gamma"].reshape(Cout, 1)
    beta = params["beta"].reshape(Cout, 1)

    emat = _build_scatter(H, W)
    tmask = _build_tap_mask(H2, W2)

    cparams = pltpu.CompilerParams(
        dimension_semantics=("parallel",),
        vmem_limit_bytes=48 * 1024 * 1024,
    )

    NB1 = 8 if B % 16 == 0 else 1
    kernel1 = functools.partial(_unet_fused_kernel, W2=W2, Co=Co, Cout=Cout,
                                NB=NB1)
    h2, psum, psq = pl.pallas_call(
        kernel1,
        out_shape=(jax.ShapeDtypeStruct((B, Cout, HW2), jnp.bfloat16),
                   jax.ShapeDtypeStruct((B, Cout, 1), jnp.float32),
                   jax.ShapeDtypeStruct((B, Cout, 1), jnp.float32)),
        grid=(B // NB1,),
        in_specs=[
            pl.BlockSpec((NB1, Cup, HW), lambda b: (b, 0, 0)),
            pl.BlockSpec((NB1, Cx, HW2), lambda b: (b, 0, 0)),
            pl.BlockSpec((4 * Co, Cup), lambda b: (0, 0)),
            pl.BlockSpec((Co, 1), lambda b: (0, 0)),
            pl.BlockSpec((4, HW, HW2), lambda b: (0, 0, 0)),
            pl.BlockSpec((9 * Cout, Cmid), lambda b: (0, 0)),
            pl.BlockSpec((Cout, 1), lambda b: (0, 0)),
            pl.BlockSpec((9 * Cout, Cout), lambda b: (0, 0)),
            pl.BlockSpec((Cout, 1), lambda b: (0, 0)),
            pl.BlockSpec((9, HW2), lambda b: (0, 0)),
        ],
        out_specs=(pl.BlockSpec((NB1, Cout, HW2), lambda b: (b, 0, 0)),
                   pl.BlockSpec((NB1, Cout, 1), lambda b: (b, 0, 0)),
                   pl.BlockSpec((NB1, Cout, 1), lambda b: (b, 0, 0))),
        compiler_params=cparams,
    )(u_flat, x_flat, wstack, bup, emat, w1_all, b1, w2_all, b2, tmask)

    count = B * HW2
    mean = jnp.sum(psum, axis=0) / count
    var = jnp.sum(psq, axis=0) / count - mean * mean
    scale = gamma * jax.lax.rsqrt(var + eps)
    shift = beta - mean * scale

    NB = 4 if B % 4 == 0 else 1
    out = pl.pallas_call(
        _bn_apply_kernel,
        out_shape=jax.ShapeDtypeStruct((B, Cout, HW2), jnp.float32),
        grid=(B // NB,),
        in_specs=[pl.BlockSpec((NB, Cout, HW2), lambda b: (b, 0, 0)),
                  pl.BlockSpec((Cout, 1), lambda b: (0, 0)),
                  pl.BlockSpec((Cout, 1), lambda b: (0, 0))],
        out_specs=pl.BlockSpec((NB, Cout, HW2), lambda b: (b, 0, 0)),
        compiler_params=cparams,
    )(h2, scale, shift)

    return out.reshape(B, Cout, H2, W2)


def kernel(up_in, x_in, w_up, b_up, w1, b1, w2, b2, gamma, beta):
    params = {"w_up": w_up, "b_up": b_up, "w1": w1, "b1": b1,
              "w2": w2, "b2": b2, "gamma": gamma, "beta": beta}
    return _unet_block_forward(up_in, x_in, params)


# speedup vs baseline: 1.5795x; 1.2686x over previous
import functools

import numpy as np
import jax
import jax.numpy as jnp
from jax.experimental import pallas as pl
from jax.experimental.pallas import tpu as pltpu


def _build_tap_mask(H2, W2):
    m = np.zeros((9, H2 * W2), np.float32)
    for dh in range(3):
        for dw in range(3):
            t = dh * 3 + dw
            ii = np.arange(H2)[:, None] + dh - 1
            jj = np.arange(W2)[None, :] + dw - 1
            valid = (ii >= 0) & (ii < H2) & (jj >= 0) & (jj < W2)
            m[t] = valid.astype(np.float32).reshape(-1)
    return jnp.asarray(m)


def _build_scatter(H, W):
    H2, W2 = 2 * H, 2 * W
    E = np.zeros((4, H * W, H2 * W2), np.float32)
    for i in range(H2):
        for j in range(W2):
            p = (i % 2) * 2 + (j % 2)
            E[p, (i // 2) * W + (j // 2), i * W2 + j] = 1.0
    return jnp.asarray(E.reshape(4 * H * W, H2 * W2), jnp.bfloat16)


def _unet_fused_kernel(u_ref, x_ref, wstack_ref, bup_ref, e_ref,
                       w1_ref, b1_ref, w2_ref, b2_ref, tmask_ref,
                       out_ref, sum_ref, ssq_ref, *, W2, Co, Cout, NB):
    HW2 = x_ref.shape[-1]
    masks = tmask_ref[...]

    def conv3x3_relu(xb, w_all, b_col, C):
        y = jnp.dot(w_all, xb,
                    preferred_element_type=jnp.float32)
        yb = y.astype(jnp.bfloat16)
        acc = y[4 * C:5 * C] + b_col
        for t in range(9):
            if t == 4:
                continue
            dh, dw = t // 3, t % 3
            off = (dh - 1) * W2 + (dw - 1)
            packed = pltpu.bitcast(yb[t * C:(t + 1) * C], jnp.float32)
            rolled = pltpu.roll(packed, shift=(-off) % HW2, axis=1)
            kept = jnp.where(masks[t:t + 1, :] > 0.5, rolled, 0.0)
            acc = acc + pltpu.bitcast(kept,
                                      jnp.bfloat16).astype(jnp.float32)
        return jnp.maximum(acc, 0.0)

    for i in range(NB):
        ub = u_ref[i].astype(jnp.bfloat16)
        z = jnp.dot(wstack_ref[...], ub,
                    preferred_element_type=jnp.float32)
        zb = z.astype(jnp.bfloat16)
        zcat = jnp.concatenate([zb[p * Co:(p + 1) * Co] for p in range(4)],
                               axis=1)
        up = jnp.dot(zcat, e_ref[...],
                     preferred_element_type=jnp.float32)
        up = up + bup_ref[...]

        cat_b = jnp.concatenate([up.astype(jnp.bfloat16),
                                 x_ref[i].astype(jnp.bfloat16)], axis=0)

        h1 = conv3x3_relu(cat_b, w1_ref[...], b1_ref[...], Cout)
        h2 = conv3x3_relu(h1.astype(jnp.bfloat16), w2_ref[...], b2_ref[...],
                          Cout)

        out_ref[i] = h2.astype(jnp.bfloat16)
        sum_ref[i] = jnp.sum(h2, axis=1, keepdims=True)
        ssq_ref[i] = jnp.sum(h2 * h2, axis=1, keepdims=True)


def _bn_apply_kernel(h_ref, scale_ref, shift_ref, out_ref):
    out_ref[...] = (h_ref[...].astype(jnp.float32) * scale_ref[...]
                    + shift_ref[...])


@jax.jit
def _unet_block_forward(up_in, x_in, params):
    B, Cup, H, W = up_in.shape
    Co = Cup // 2
    _, Cx, H2, W2 = x_in.shape
    assert H2 == 2 * H and W2 == 2 * W
    Cmid = Cx + Co
    Cout = Cmid // 2
    HW = H * W
    HW2 = H2 * W2
    eps = 1e-5
    assert Cup % 8 == 0 and Co % 8 == 0 and Cx % 8 == 0 and Cout % 8 == 0

    u_flat = up_in.reshape(B, Cup, HW)
    x_flat = x_in.reshape(B, Cx, HW2)

    wstack = jnp.transpose(params["w_up"], (2, 3, 1, 0)) \
        .reshape(4 * Co, Cup).astype(jnp.bfloat16)
    bup = params["b_up"].reshape(Co, 1)

    w1_all = jnp.transpose(params["w1"], (2, 3, 0, 1)) \
        .reshape(9 * Cout, Cmid).astype(jnp.bfloat16)
    b1 = params["b1"].reshape(Cout, 1)
    w2_all = jnp.transpose(params["w2"], (2, 3, 0, 1)) \
        .reshape(9 * Cout, Cout).astype(jnp.bfloat16)
    b2 = params["b2"].reshape(Cout, 1)
    gamma = params["gamma"].reshape(Cout, 1)
    beta = params["beta"].reshape(Cout, 1)

    emat = _build_scatter(H, W)
    tmask = _build_tap_mask(H2, W2)

    cparams = pltpu.CompilerParams(
        dimension_semantics=("parallel",),
        vmem_limit_bytes=48 * 1024 * 1024,
    )

    NB1 = 4 if B % 8 == 0 else 1
    kernel1 = functools.partial(_unet_fused_kernel, W2=W2, Co=Co, Cout=Cout,
                                NB=NB1)
    h2, psum, psq = pl.pallas_call(
        kernel1,
        out_shape=(jax.ShapeDtypeStruct((B, Cout, HW2), jnp.bfloat16),
                   jax.ShapeDtypeStruct((B, Cout, 1), jnp.float32),
                   jax.ShapeDtypeStruct((B, Cout, 1), jnp.float32)),
        grid=(B // NB1,),
        in_specs=[
            pl.BlockSpec((NB1, Cup, HW), lambda b: (b, 0, 0)),
            pl.BlockSpec((NB1, Cx, HW2), lambda b: (b, 0, 0)),
            pl.BlockSpec((4 * Co, Cup), lambda b: (0, 0)),
            pl.BlockSpec((Co, 1), lambda b: (0, 0)),
            pl.BlockSpec((4 * HW, HW2), lambda b: (0, 0)),
            pl.BlockSpec((9 * Cout, Cmid), lambda b: (0, 0)),
            pl.BlockSpec((Cout, 1), lambda b: (0, 0)),
            pl.BlockSpec((9 * Cout, Cout), lambda b: (0, 0)),
            pl.BlockSpec((Cout, 1), lambda b: (0, 0)),
            pl.BlockSpec((9, HW2), lambda b: (0, 0)),
        ],
        out_specs=(pl.BlockSpec((NB1, Cout, HW2), lambda b: (b, 0, 0)),
                   pl.BlockSpec((NB1, Cout, 1), lambda b: (b, 0, 0)),
                   pl.BlockSpec((NB1, Cout, 1), lambda b: (b, 0, 0))),
        compiler_params=cparams,
    )(u_flat, x_flat, wstack, bup, emat, w1_all, b1, w2_all, b2, tmask)

    count = B * HW2
    mean = jnp.sum(psum, axis=0) / count
    var = jnp.sum(psq, axis=0) / count - mean * mean
    scale = gamma * jax.lax.rsqrt(var + eps)
    shift = beta - mean * scale

    NB = 4 if B % 4 == 0 else 1
    out = pl.pallas_call(
        _bn_apply_kernel,
        out_shape=jax.ShapeDtypeStruct((B, Cout, HW2), jnp.float32),
        grid=(B // NB,),
        in_specs=[pl.BlockSpec((NB, Cout, HW2), lambda b: (b, 0, 0)),
                  pl.BlockSpec((Cout, 1), lambda b: (0, 0)),
                  pl.BlockSpec((Cout, 1), lambda b: (0, 0))],
        out_specs=pl.BlockSpec((NB, Cout, HW2), lambda b: (b, 0, 0)),
        compiler_params=cparams,
    )(h2, scale, shift)

    return out.reshape(B, Cout, H2, W2)


def kernel(up_in, x_in, w_up, b_up, w1, b1, w2, b2, gamma, beta):
    params = {"w_up": w_up, "b_up": b_up, "w1": w1, "b1": b1,
              "w2": w2, "b2": b2, "gamma": gamma, "beta": beta}
    return _unet_block_forward(up_in, x_in, params)
